# baseline (speedup 1.0000x reference)
import functools

import jax
import jax.numpy as jnp
from jax import lax
from jax.experimental import pallas as pl
from jax.experimental.pallas import tpu as pltpu
from jax.experimental.pallas import tpu_sc as plsc

_N = 10000
_E = 320000
_NW = 32
_EPW = _E // _NW
_NP = 10240
_RPT = _NP // 16


def _make_edge_agg(width, chunk, nbuf, with_deg=False, out_width=None):
    ow = out_width or width
    nchunk = _EPW // chunk
    groups = nchunk // nbuf
    tail = nchunk % nbuf
    mesh = plsc.VectorSubcoreMesh(core_axis_name="c", subcore_axis_name="s")

    out_type = [jax.ShapeDtypeStruct((2, _NP, ow), jnp.float32)]
    scratch = [
        pltpu.VMEM((_EPW,), jnp.int32),
        pltpu.VMEM((_EPW,), jnp.int32),
        pltpu.VMEM((nbuf, chunk, width), jnp.float32),
        pltpu.VMEM_SHARED((_NP, width), jnp.float32),
        pltpu.SemaphoreType.DMA((nbuf,)),
        pltpu.SemaphoreType.DMA((nbuf,)),
    ]
    if with_deg:
        out_type.append(jax.ShapeDtypeStruct((2, _NP // 16, 16), jnp.float32))
        scratch += [
            pltpu.VMEM((_NP // 16, 16), jnp.float32),
            pltpu.VMEM((_NP // 16,), jnp.int32),
            pltpu.VMEM_SHARED((_NP // 16, 16), jnp.float32),
        ]

    @functools.partial(
        pl.kernel,
        mesh=mesh,
        out_type=tuple(out_type),
        scratch_types=scratch,
        compiler_params=pltpu.CompilerParams(use_tc_tiling_on_sc=False,
                                             needs_layout_passes=False),
    )
    def agg_kernel(z_hbm, ei_hbm, zeros_hbm, *rest):
        if with_deg:
            (zeros16_hbm, out_hbm, dout_hbm, src_v, dst_v, rows_v, acc_sh,
             gsem, ssem, hist_v, idv, dacc_sh) = rest
        else:
            out_hbm, src_v, dst_v, rows_v, acc_sh, gsem, ssem = rest
        cid = lax.axis_index("c")
        sid = lax.axis_index("s")
        wid = sid * 2 + cid
        nh = _NP // 16
        nht = nh // 16
        pltpu.sync_copy(zeros_hbm, acc_sh.at[pl.ds(sid * _RPT, _RPT)])
        pltpu.sync_copy(ei_hbm.at[0, pl.ds(wid * _EPW, _EPW)], src_v)
        pltpu.sync_copy(ei_hbm.at[1, pl.ds(wid * _EPW, _EPW)], dst_v)
        if with_deg:
            pltpu.sync_copy(zeros16_hbm,
                            dacc_sh.at[pl.ds(sid * nht, nht)])
        plsc.subcore_barrier()

        def sidx(j):
            return src_v.at[pl.ds(j * chunk, chunk)]

        def didx(j):
            return dst_v.at[pl.ds(j * chunk, chunk)]

        def start_gather(j, b):
            pltpu.async_copy(z_hbm.at[sidx(j)], rows_v.at[b], gsem.at[b])

        def wait_gather(j, b):
            pltpu.make_async_copy(z_hbm.at[sidx(j)], rows_v.at[b],
                                  gsem.at[b]).wait()

        for b in range(nbuf):
            start_gather(b, b)

        if with_deg:
            ones16 = jnp.full((16,), 1.0, jnp.float32)
            iota16 = lax.iota(jnp.int32, 16)

            def zero_hist(i, carry):
                hist_v[i] = jnp.zeros((16,), jnp.float32)
                return carry

            lax.fori_loop(0, nh, zero_hist, 0, unroll=False)
            for k in range(nh // 16):
                idv[pl.ds(k * 16, 16)] = iota16 + k * 16

            def hist_step(i, carry):
                d16 = dst_v[pl.ds(i * 16, 16)]
                plsc.addupdate_scatter(
                    hist_v, [lax.shift_right_logical(d16, 4),
                             lax.bitwise_and(d16, 15)], ones16)
                return carry

            lax.fori_loop(0, _EPW // 16, hist_step, 0, unroll=False)
            for k in range(nh // 128):
                pltpu.sync_copy(hist_v.at[pl.ds(k * 128, 128)],
                                dacc_sh.at[idv.at[pl.ds(k * 128, 128)]],
                                add=True)

        def group(g, carry):
            for b in range(nbuf):
                j = g * nbuf + b
                wait_gather(j, b)
                pltpu.async_copy(rows_v.at[b], acc_sh.at[didx(j)],
                                 ssem.at[b], add=True)
                pltpu.make_async_copy(rows_v.at[b], acc_sh.at[didx(j)],
                                      ssem.at[b]).wait()

                @pl.when(j + nbuf < nchunk)
                def _():
                    start_gather(j + nbuf, b)
            return carry

        lax.fori_loop(0, groups, group, 0, unroll=False)
        for t in range(tail):
            j = groups * nbuf + t
            wait_gather(j, t)
            pltpu.sync_copy(rows_v.at[t], acc_sh.at[didx(j)], add=True)
        plsc.subcore_barrier()
        rows = pl.ds(sid * _RPT, _RPT)
        if ow == width:
            pltpu.sync_copy(acc_sh.at[rows], out_hbm.at[cid, rows])
        else:
            pltpu.sync_copy(acc_sh.at[rows],
                            out_hbm.at[cid, rows, pl.ds(0, width)])
        if with_deg:
            hrows = pl.ds(sid * nht, nht)
            pltpu.sync_copy(dacc_sh.at[hrows], dout_hbm.at[cid, hrows])

    return agg_kernel


_agg0 = _make_edge_agg(128, 40, 3, with_deg=True)
_agg1 = _make_edge_agg(128, 40, 5)
_agg2 = _make_edge_agg(64, 80, 6, out_width=128)


def _tc_z0(x_ref, wn_ref, out_ref):
    out_ref[...] = jnp.dot(x_ref[...], wn_ref[...],
                           preferred_element_type=jnp.float32)


def _tc_layer0(x_ref, agg_ref, inv_ref, ws_ref, bs_ref, bn_ref, g_ref, be_ref,
               wn1_ref, h1_ref, z1_ref):
    agg = agg_ref[0, :_N] + agg_ref[1, :_N]
    pre = (jnp.dot(x_ref[...], ws_ref[...], preferred_element_type=jnp.float32)
           + bs_ref[...] + bn_ref[...] + agg * inv_ref[...])
    mu = jnp.mean(pre, axis=0, keepdims=True)
    var = jnp.mean((pre - mu) * (pre - mu), axis=0, keepdims=True)
    h = g_ref[...] * (pre - mu) * lax.rsqrt(var + 1e-5) + be_ref[...]
    h = jnp.maximum(h, 0.0)
    h1_ref[...] = h
    z1_ref[...] = jnp.dot(h, wn1_ref[...], preferred_element_type=jnp.float32)


def _tc_layer1(h1_ref, agg_ref, inv_ref, ws_ref, bs_ref, bn_ref, g_ref, be_ref,
               wn2_ref, h2_ref, z2_ref):
    agg = agg_ref[0, :_N] + agg_ref[1, :_N]
    pre = (jnp.dot(h1_ref[...], ws_ref[...], preferred_element_type=jnp.float32)
           + bs_ref[...] + bn_ref[...] + agg * inv_ref[...])
    mu = jnp.mean(pre, axis=0, keepdims=True)
    var = jnp.mean((pre - mu) * (pre - mu), axis=0, keepdims=True)
    h = g_ref[...] * (pre - mu) * lax.rsqrt(var + 1e-5) + be_ref[...]
    h = jnp.maximum(h, 0.0)
    h2_ref[...] = h
    z2_ref[...] = jnp.dot(h, wn2_ref[...], preferred_element_type=jnp.float32)


def _tc_layer2(h2_ref, agg_ref, inv_ref, ws_ref, bs_ref, bn_ref, out_ref):
    agg = agg_ref[0, :_N, :64] + agg_ref[1, :_N, :64]
    out_ref[...] = (jnp.dot(h2_ref[...], ws_ref[...],
                            preferred_element_type=jnp.float32)
                    + bs_ref[...] + bn_ref[...] + agg * inv_ref[...])


def kernel(x, edge_index, Ws0, bs0, Wn0, bn0, g0, be0,
           Ws1, bs1, Wn1, bn1, g1, be1, Ws2, bs2, Wn2, bn2):
    f32 = jnp.float32
    zeros128 = jnp.zeros((_RPT, 128), f32)

    z0 = pl.pallas_call(
        _tc_z0,
        out_shape=jax.ShapeDtypeStruct((_N, 128), f32),
    )(x, Wn0)

    agg0, deg0 = _agg0(z0, edge_index, zeros128,
                       jnp.zeros((_NP // 16 // 16, 16), f32))

    deg = (deg0[0] + deg0[1]).reshape(_NP, 1)[:_N]
    invd = 1.0 / jnp.maximum(deg, 1.0)

    h1, z1 = pl.pallas_call(
        _tc_layer0,
        out_shape=(
            jax.ShapeDtypeStruct((_N, 128), f32),
            jax.ShapeDtypeStruct((_N, 128), f32),
        ),
    )(x, agg0, invd, Ws0, bs0.reshape(1, 128), bn0.reshape(1, 128),
      g0.reshape(1, 128), be0.reshape(1, 128), Wn1)

    (agg1,) = _agg1(z1, edge_index, zeros128)

    h2, z2 = pl.pallas_call(
        _tc_layer1,
        out_shape=(
            jax.ShapeDtypeStruct((_N, 128), f32),
            jax.ShapeDtypeStruct((_N, 64), f32),
        ),
    )(h1, agg1, invd, Ws1, bs1.reshape(1, 128), bn1.reshape(1, 128),
      g1.reshape(1, 128), be1.reshape(1, 128), Wn2)

    (agg2,) = _agg2(z2, edge_index, jnp.zeros((_RPT, 64), f32))

    out = pl.pallas_call(
        _tc_layer2,
        out_shape=jax.ShapeDtypeStruct((_N, 64), f32),
    )(h2, agg2, invd, Ws2, bs2.reshape(1, 64), bn2.reshape(1, 64))

    return out


# speedup vs baseline: 1.0712x; 1.0712x over previous
import functools

import jax
import jax.numpy as jnp
from jax import lax
from jax.experimental import pallas as pl
from jax.experimental.pallas import tpu as pltpu
from jax.experimental.pallas import tpu_sc as plsc

_N = 10000
_E = 320000
_NW = 32
_EPW = _E // _NW
_NP = 10240
_RPT = _NP // 16


def _make_edge_agg(width, chunk, nbuf, with_deg=False, out_width=None):
    ow = out_width or width
    nchunk = _EPW // chunk
    groups = nchunk // nbuf
    tail = nchunk % nbuf
    mesh = plsc.VectorSubcoreMesh(core_axis_name="c", subcore_axis_name="s")

    out_type = [jax.ShapeDtypeStruct((2, _NP, ow), jnp.float32)]
    scratch = [
        pltpu.VMEM((_EPW,), jnp.int32),
        pltpu.VMEM((_EPW,), jnp.int32),
        pltpu.VMEM((nbuf, chunk, width), jnp.float32),
        pltpu.VMEM_SHARED((_NP, width), jnp.float32),
        pltpu.SemaphoreType.DMA((nbuf,)),
        pltpu.SemaphoreType.DMA((nbuf,)),
    ]
    if with_deg:
        out_type.append(jax.ShapeDtypeStruct((2, _NP // 16, 16), jnp.float32))
        scratch += [
            pltpu.VMEM((_NP // 16, 16), jnp.float32),
            pltpu.VMEM((_NP // 16,), jnp.int32),
            pltpu.VMEM_SHARED((_NP // 16, 16), jnp.float32),
        ]

    @functools.partial(
        pl.kernel,
        mesh=mesh,
        out_type=tuple(out_type),
        scratch_types=scratch,
        compiler_params=pltpu.CompilerParams(use_tc_tiling_on_sc=False,
                                             needs_layout_passes=False),
    )
    def agg_kernel(z_hbm, ei_hbm, zeros_hbm, *rest):
        if with_deg:
            (zeros16_hbm, out_hbm, dout_hbm, src_v, dst_v, rows_v, acc_sh,
             gsem, ssem, hist_v, idv, dacc_sh) = rest
        else:
            out_hbm, src_v, dst_v, rows_v, acc_sh, gsem, ssem = rest
        cid = lax.axis_index("c")
        sid = lax.axis_index("s")
        wid = sid * 2 + cid
        nh = _NP // 16
        nht = nh // 16
        pltpu.sync_copy(zeros_hbm, acc_sh.at[pl.ds(sid * _RPT, _RPT)])
        pltpu.sync_copy(ei_hbm.at[0, pl.ds(wid * _EPW, _EPW)], src_v)
        pltpu.sync_copy(ei_hbm.at[1, pl.ds(wid * _EPW, _EPW)], dst_v)
        if with_deg:
            pltpu.sync_copy(zeros16_hbm,
                            dacc_sh.at[pl.ds(sid * nht, nht)])
        plsc.subcore_barrier()

        def sidx(j):
            return src_v.at[pl.ds(j * chunk, chunk)]

        def didx(j):
            return dst_v.at[pl.ds(j * chunk, chunk)]

        def start_gather(j, b):
            pltpu.async_copy(z_hbm.at[sidx(j)], rows_v.at[b], gsem.at[b])

        def wait_gather(j, b):
            pltpu.make_async_copy(z_hbm.at[sidx(j)], rows_v.at[b],
                                  gsem.at[b]).wait()

        for b in range(nbuf):
            start_gather(b, b)

        if with_deg:
            ones16 = jnp.full((16,), 1.0, jnp.float32)
            iota16 = lax.iota(jnp.int32, 16)

            def zero_hist(i, carry):
                hist_v[i] = jnp.zeros((16,), jnp.float32)
                return carry

            lax.fori_loop(0, nh, zero_hist, 0, unroll=False)
            for k in range(nh // 16):
                idv[pl.ds(k * 16, 16)] = iota16 + k * 16

            def hist_step(i, carry):
                d16 = dst_v[pl.ds(i * 16, 16)]
                plsc.addupdate_scatter(
                    hist_v, [lax.shift_right_logical(d16, 4),
                             lax.bitwise_and(d16, 15)], ones16)
                return carry

            lax.fori_loop(0, _EPW // 16, hist_step, 0, unroll=False)
            for k in range(nh // 128):
                pltpu.sync_copy(hist_v.at[pl.ds(k * 128, 128)],
                                dacc_sh.at[idv.at[pl.ds(k * 128, 128)]],
                                add=True)

        def group(g, carry):
            for b in range(nbuf):
                j = g * nbuf + b
                wait_gather(j, b)
                pltpu.async_copy(rows_v.at[b], acc_sh.at[didx(j)],
                                 ssem.at[b], add=True)
                pltpu.make_async_copy(rows_v.at[b], acc_sh.at[didx(j)],
                                      ssem.at[b]).wait()

                @pl.when(j + nbuf < nchunk)
                def _():
                    start_gather(j + nbuf, b)
            return carry

        lax.fori_loop(0, groups, group, 0, unroll=False)
        for t in range(tail):
            j = groups * nbuf + t
            wait_gather(j, t)
            pltpu.sync_copy(rows_v.at[t], acc_sh.at[didx(j)], add=True)
        plsc.subcore_barrier()
        rows = pl.ds(sid * _RPT, _RPT)
        if ow == width:
            pltpu.sync_copy(acc_sh.at[rows], out_hbm.at[cid, rows])
        else:
            pltpu.sync_copy(acc_sh.at[rows],
                            out_hbm.at[cid, rows, pl.ds(0, width)])
        if with_deg:
            hrows = pl.ds(sid * nht, nht)
            pltpu.sync_copy(dacc_sh.at[hrows], dout_hbm.at[cid, hrows])

    return agg_kernel


def _make_edge_agg0():
    chunk, nbuf, seg_e, nseg = 40, 5, 2000, 5
    seg_c = seg_e // chunk
    sgroups = seg_c // nbuf
    mesh = plsc.VectorSubcoreMesh(core_axis_name="c", subcore_axis_name="s")

    @functools.partial(
        pl.kernel,
        mesh=mesh,
        out_type=(
            jax.ShapeDtypeStruct((2, _NP, 128), jnp.float32),
            jax.ShapeDtypeStruct((2, _NP // 16, 16), jnp.float32),
        ),
        scratch_types=[
            pltpu.VMEM((2, seg_e), jnp.int32),
            pltpu.VMEM((2, seg_e), jnp.int32),
            pltpu.VMEM((nbuf, chunk, 128), jnp.float32),
            pltpu.VMEM((_NP // 16, 16), jnp.float32),
            pltpu.VMEM((_NP // 16,), jnp.int32),
            pltpu.VMEM_SHARED((_NP, 128), jnp.float32),
            pltpu.VMEM_SHARED((_NP // 16, 16), jnp.float32),
            pltpu.SemaphoreType.DMA((nbuf,)),
            pltpu.SemaphoreType.DMA((nbuf,)),
            pltpu.SemaphoreType.DMA,
        ],
        compiler_params=pltpu.CompilerParams(use_tc_tiling_on_sc=False,
                                             needs_layout_passes=False),
    )
    def agg_kernel(z_hbm, ei_hbm, zeros_hbm, zeros16_hbm, out_hbm, dout_hbm,
                   sbank, dbank, rows_v, hist_v, idv, acc_sh, dacc_sh,
                   gsem, ssem, isem):
        cid = lax.axis_index("c")
        sid = lax.axis_index("s")
        wid = sid * 2 + cid
        base = wid * _EPW
        nh = _NP // 16
        nht = nh // 16
        pltpu.sync_copy(zeros_hbm, acc_sh.at[pl.ds(sid * _RPT, _RPT)])
        pltpu.sync_copy(zeros16_hbm, dacc_sh.at[pl.ds(sid * nht, nht)])
        pltpu.sync_copy(ei_hbm.at[0, pl.ds(base, seg_e)], sbank.at[0])
        pltpu.sync_copy(ei_hbm.at[1, pl.ds(base, seg_e)], dbank.at[0])
        plsc.subcore_barrier()

        def idx_descs(seg, bank):
            off = base + seg * seg_e
            return (
                pltpu.make_async_copy(ei_hbm.at[0, pl.ds(off, seg_e)],
                                      sbank.at[bank], isem),
                pltpu.make_async_copy(ei_hbm.at[1, pl.ds(off, seg_e)],
                                      dbank.at[bank], isem),
            )

        def sidx(bank, jl):
            return sbank.at[bank, pl.ds(jl * chunk, chunk)]

        def didx(bank, jl):
            return dbank.at[bank, pl.ds(jl * chunk, chunk)]

        def start_gather(bank, jl, b):
            pltpu.async_copy(z_hbm.at[sidx(bank, jl)], rows_v.at[b],
                             gsem.at[b])

        def wait_gather(bank, jl, b):
            pltpu.make_async_copy(z_hbm.at[sidx(bank, jl)], rows_v.at[b],
                                  gsem.at[b]).wait()

        def slot(bank, jl, b):
            wait_gather(bank, jl, b)
            pltpu.async_copy(rows_v.at[b], acc_sh.at[didx(bank, jl)],
                             ssem.at[b], add=True)
            pltpu.make_async_copy(rows_v.at[b], acc_sh.at[didx(bank, jl)],
                                  ssem.at[b]).wait()

        ones16 = jnp.full((16,), 1.0, jnp.float32)
        iota16 = lax.iota(jnp.int32, 16)

        def zero_hist(i, carry):
            hist_v[i] = jnp.zeros((16,), jnp.float32)
            return carry

        lax.fori_loop(0, nh, zero_hist, 0, unroll=False)
        for k in range(nh // 16):
            idv[pl.ds(k * 16, 16)] = iota16 + k * 16

        for b in range(nbuf):
            start_gather(0, b, b)

        for s in range(nseg):
            cur = s % 2
            nxt = (s + 1) % 2

            def hist_step(i, carry):
                d16 = dbank[cur, pl.ds(i * 16, 16)]
                plsc.addupdate_scatter(
                    hist_v, [lax.shift_right_logical(d16, 4),
                             lax.bitwise_and(d16, 15)], ones16)
                return carry

            lax.fori_loop(0, seg_e // 16, hist_step, 0, unroll=False)

            for b in range(nbuf):
                slot(cur, b, b)
                start_gather(cur, b + nbuf, b)
            if s + 1 < nseg:
                for d in idx_descs(s + 1, nxt):
                    d.start()

            def group(g, carry):
                for b in range(nbuf):
                    jl = g * nbuf + b
                    slot(cur, jl, b)
                    start_gather(cur, jl + nbuf, b)
                return carry

            lax.fori_loop(1, sgroups - 1, group, 0, unroll=False)
            if s + 1 < nseg:
                for d in idx_descs(s + 1, nxt):
                    d.wait()
            for b in range(nbuf):
                jl = (sgroups - 1) * nbuf + b
                slot(cur, jl, b)
                if s + 1 < nseg:
                    start_gather(nxt, b, b)

        for k in range(nh // 128):
            pltpu.sync_copy(hist_v.at[pl.ds(k * 128, 128)],
                            dacc_sh.at[idv.at[pl.ds(k * 128, 128)]],
                            add=True)
        plsc.subcore_barrier()
        rows = pl.ds(sid * _RPT, _RPT)
        pltpu.sync_copy(acc_sh.at[rows], out_hbm.at[cid, rows])
        hrows = pl.ds(sid * nht, nht)
        pltpu.sync_copy(dacc_sh.at[hrows], dout_hbm.at[cid, hrows])

    return agg_kernel


_agg0 = _make_edge_agg0()
_agg1 = _make_edge_agg(128, 40, 5)
_agg2 = _make_edge_agg(64, 80, 8, out_width=128)


def _tc_z0(x_ref, wn_ref, out_ref):
    out_ref[...] = jnp.dot(x_ref[...], wn_ref[...],
                           preferred_element_type=jnp.float32)


def _tc_layer0(x_ref, agg_ref, inv_ref, ws_ref, bs_ref, bn_ref, g_ref, be_ref,
               wn1_ref, h1_ref, z1_ref):
    agg = agg_ref[0, :_N] + agg_ref[1, :_N]
    pre = (jnp.dot(x_ref[...], ws_ref[...], preferred_element_type=jnp.float32)
           + bs_ref[...] + bn_ref[...] + agg * inv_ref[...])
    mu = jnp.mean(pre, axis=0, keepdims=True)
    var = jnp.mean((pre - mu) * (pre - mu), axis=0, keepdims=True)
    h = g_ref[...] * (pre - mu) * lax.rsqrt(var + 1e-5) + be_ref[...]
    h = jnp.maximum(h, 0.0)
    h1_ref[...] = h
    z1_ref[...] = jnp.dot(h, wn1_ref[...], preferred_element_type=jnp.float32)


def _tc_layer1(h1_ref, agg_ref, inv_ref, ws_ref, bs_ref, bn_ref, g_ref, be_ref,
               wn2_ref, h2_ref, z2_ref):
    agg = agg_ref[0, :_N] + agg_ref[1, :_N]
    pre = (jnp.dot(h1_ref[...], ws_ref[...], preferred_element_type=jnp.float32)
           + bs_ref[...] + bn_ref[...] + agg * inv_ref[...])
    mu = jnp.mean(pre, axis=0, keepdims=True)
    var = jnp.mean((pre - mu) * (pre - mu), axis=0, keepdims=True)
    h = g_ref[...] * (pre - mu) * lax.rsqrt(var + 1e-5) + be_ref[...]
    h = jnp.maximum(h, 0.0)
    h2_ref[...] = h
    z2_ref[...] = jnp.dot(h, wn2_ref[...], preferred_element_type=jnp.float32)


def _tc_layer2(h2_ref, agg_ref, inv_ref, ws_ref, bs_ref, bn_ref, out_ref):
    agg = agg_ref[0, :_N, :64] + agg_ref[1, :_N, :64]
    out_ref[...] = (jnp.dot(h2_ref[...], ws_ref[...],
                            preferred_element_type=jnp.float32)
                    + bs_ref[...] + bn_ref[...] + agg * inv_ref[...])


def kernel(x, edge_index, Ws0, bs0, Wn0, bn0, g0, be0,
           Ws1, bs1, Wn1, bn1, g1, be1, Ws2, bs2, Wn2, bn2):
    f32 = jnp.float32
    zeros128 = jnp.zeros((_RPT, 128), f32)

    z0 = pl.pallas_call(
        _tc_z0,
        out_shape=jax.ShapeDtypeStruct((_N, 128), f32),
    )(x, Wn0)

    agg0, deg0 = _agg0(z0, edge_index, zeros128,
                       jnp.zeros((_NP // 16 // 16, 16), f32))

    deg = (deg0[0] + deg0[1]).reshape(_NP, 1)[:_N]
    invd = 1.0 / jnp.maximum(deg, 1.0)

    h1, z1 = pl.pallas_call(
        _tc_layer0,
        out_shape=(
            jax.ShapeDtypeStruct((_N, 128), f32),
            jax.ShapeDtypeStruct((_N, 128), f32),
        ),
    )(x, agg0, invd, Ws0, bs0.reshape(1, 128), bn0.reshape(1, 128),
      g0.reshape(1, 128), be0.reshape(1, 128), Wn1)

    (agg1,) = _agg1(z1, edge_index, zeros128)

    h2, z2 = pl.pallas_call(
        _tc_layer1,
        out_shape=(
            jax.ShapeDtypeStruct((_N, 128), f32),
            jax.ShapeDtypeStruct((_N, 64), f32),
        ),
    )(h1, agg1, invd, Ws1, bs1.reshape(1, 128), bn1.reshape(1, 128),
      g1.reshape(1, 128), be1.reshape(1, 128), Wn2)

    (agg2,) = _agg2(z2, edge_index, jnp.zeros((_RPT, 64), f32))

    out = pl.pallas_call(
        _tc_layer2,
        out_shape=jax.ShapeDtypeStruct((_N, 64), f32),
    )(h2, agg2, invd, Ws2, bs2.reshape(1, 64), bn2.reshape(1, 64))

    return out
